# Initial kernel scaffold; baseline (speedup 1.0000x reference)
import jax
import jax.numpy as jnp
from jax.experimental import pallas as pl


def kernel(x, edge_index, W1, b1, W2, b2):
    raise NotImplementedError("write your pallas kernel here")


# speedup vs baseline: 19.8867x; 19.8867x over previous
import functools

import jax
import jax.numpy as jnp
from jax import lax
from jax.experimental import pallas as pl
from jax.experimental.pallas import tpu as pltpu
from jax.experimental.pallas import tpu_sc as plsc

N = 10000
D = 128
E = 320000
NC = 2
NS = 16
NW = NC * NS
B = 128
CH = 80
EPW = CH * B
EP = NW * EPW
PADE = EP - E
NR = 10240
STRIPE = NR // NS

_mesh = plsc.VectorSubcoreMesh(core_axis_name="c", subcore_axis_name="s")


def _sc_degree(dstp, zrows, onesrows):
    @functools.partial(
        pl.kernel,
        out_type=jax.ShapeDtypeStruct((NC, NR, D), jnp.float32),
        mesh=_mesh,
        scratch_types=[
            pltpu.VMEM((CH, B), jnp.int32),
            pltpu.VMEM((B, D), jnp.float32),
            pltpu.VMEM_SHARED((NR, D), jnp.float32),
        ],
    )
    def k(dstp_hbm, zeros_hbm, ones_hbm, out_hbm, idx_v, ones_v, degw):
        cid = lax.axis_index("c")
        sid = lax.axis_index("s")
        wid = cid * NS + sid
        row0 = sid * STRIPE
        pltpu.sync_copy(zeros_hbm.at[pl.ds(row0, STRIPE)],
                        degw.at[pl.ds(row0, STRIPE)])
        pltpu.sync_copy(dstp_hbm.at[wid], idx_v)
        pltpu.sync_copy(ones_hbm, ones_v)
        plsc.subcore_barrier()

        def body(ch, carry):
            pltpu.sync_copy(ones_v, degw.at[idx_v.at[ch]], add=True)
            return carry

        lax.fori_loop(0, CH, body, 0)
        plsc.subcore_barrier()
        pltpu.sync_copy(degw.at[pl.ds(row0, STRIPE)],
                        out_hbm.at[cid, pl.ds(row0, STRIPE)])

    return k(dstp, zrows, onesrows)


def _sc_scatter(y, srcp, dstp, zrows):
    @functools.partial(
        pl.kernel,
        out_type=jax.ShapeDtypeStruct((NC, NR, D), jnp.float32),
        mesh=_mesh,
        scratch_types=[
            pltpu.VMEM((CH, B), jnp.int32),
            pltpu.VMEM((CH, B), jnp.int32),
            pltpu.VMEM((B, D), jnp.float32),
            pltpu.VMEM_SHARED((NR, D), jnp.float32),
            pltpu.SemaphoreType.DMA,
        ],
    )
    def k(y_hbm, srcp_hbm, dstp_hbm, z_hbm, out_hbm,
          sidx, didx, rows, acc, sem):
        cid = lax.axis_index("c")
        sid = lax.axis_index("s")
        wid = cid * NS + sid
        row0 = sid * STRIPE
        pltpu.sync_copy(z_hbm.at[pl.ds(row0, STRIPE)],
                        acc.at[pl.ds(row0, STRIPE)])
        pltpu.sync_copy(srcp_hbm.at[wid], sidx)
        pltpu.sync_copy(dstp_hbm.at[wid], didx)
        plsc.subcore_barrier()

        def body(ch, carry):
            pltpu.async_copy(y_hbm.at[sidx.at[ch]], rows, sem).wait()
            pltpu.sync_copy(rows, acc.at[didx.at[ch]], add=True)
            return carry

        lax.fori_loop(0, CH, body, 0)
        plsc.subcore_barrier()
        pltpu.sync_copy(acc.at[pl.ds(row0, STRIPE)],
                        out_hbm.at[cid, pl.ds(row0, STRIPE)])

    return k(y, srcp, dstp, zrows)


_R = 2000


def _tc_layer1(x, W1, d0, d1):
    def body(x_ref, w_ref, d0_ref, d1_ref, y_ref, dinv_ref):
        dinv = lax.rsqrt(1.0 + d0_ref[...] + d1_ref[...])
        y_ref[...] = jnp.dot(x_ref[...], w_ref[...],
                             preferred_element_type=jnp.float32) * dinv
        dinv_ref[...] = dinv

    return pl.pallas_call(
        body,
        grid=(N // _R,),
        in_specs=[
            pl.BlockSpec((_R, D), lambda i: (i, 0)),
            pl.BlockSpec((D, D), lambda i: (0, 0)),
            pl.BlockSpec((_R, 1), lambda i: (i, 0)),
            pl.BlockSpec((_R, 1), lambda i: (i, 0)),
        ],
        out_specs=[
            pl.BlockSpec((_R, D), lambda i: (i, 0)),
            pl.BlockSpec((_R, 1), lambda i: (i, 0)),
        ],
        out_shape=[
            jax.ShapeDtypeStruct((N, D), jnp.float32),
            jax.ShapeDtypeStruct((N, 1), jnp.float32),
        ],
    )(x, W1, d0, d1)


def _tc_layer2(a0, a1, y1, dinv, b1, W2):
    def body(a0_ref, a1_ref, y1_ref, dinv_ref, b_ref, w_ref, y2_ref):
        dinv = dinv_ref[...]
        h = jnp.maximum(
            dinv * (a0_ref[...] + a1_ref[...] + y1_ref[...]) + b_ref[...],
            0.0)
        y2_ref[...] = jnp.dot(h, w_ref[...],
                              preferred_element_type=jnp.float32) * dinv

    return pl.pallas_call(
        body,
        grid=(N // _R,),
        in_specs=[
            pl.BlockSpec((_R, D), lambda i: (i, 0)),
            pl.BlockSpec((_R, D), lambda i: (i, 0)),
            pl.BlockSpec((_R, D), lambda i: (i, 0)),
            pl.BlockSpec((_R, 1), lambda i: (i, 0)),
            pl.BlockSpec((1, D), lambda i: (0, 0)),
            pl.BlockSpec((D, D), lambda i: (0, 0)),
        ],
        out_specs=pl.BlockSpec((_R, D), lambda i: (i, 0)),
        out_shape=jax.ShapeDtypeStruct((N, D), jnp.float32),
    )(a0, a1, y1, dinv, b1, W2)


def _tc_final(a0, a1, y2, dinv, b2):
    def body(a0_ref, a1_ref, y2_ref, dinv_ref, b_ref, out_ref):
        out_ref[...] = jnp.maximum(
            dinv_ref[...] * (a0_ref[...] + a1_ref[...] + y2_ref[...])
            + b_ref[...], 0.0)

    return pl.pallas_call(
        body,
        grid=(N // _R,),
        in_specs=[
            pl.BlockSpec((_R, D), lambda i: (i, 0)),
            pl.BlockSpec((_R, D), lambda i: (i, 0)),
            pl.BlockSpec((_R, D), lambda i: (i, 0)),
            pl.BlockSpec((_R, 1), lambda i: (i, 0)),
            pl.BlockSpec((1, D), lambda i: (0, 0)),
        ],
        out_specs=pl.BlockSpec((_R, D), lambda i: (i, 0)),
        out_shape=jax.ShapeDtypeStruct((N, D), jnp.float32),
    )(a0, a1, y2, dinv, b2)


def kernel(x, edge_index, W1, b1, W2, b2):
    src = edge_index[0].astype(jnp.int32)
    dst = edge_index[1].astype(jnp.int32)

    pad_src = (jnp.arange(PADE, dtype=jnp.int32) * 13) % N
    pad_dst = N + (jnp.arange(PADE, dtype=jnp.int32) % (NR - N))
    srcp = jnp.concatenate([src, pad_src]).reshape(NW, CH, B)
    dstp = jnp.concatenate([dst, pad_dst]).reshape(NW, CH, B)

    onesrows = jnp.ones((B, D), jnp.float32)
    zrows = jnp.zeros((NR, D), jnp.float32)

    degp = _sc_degree(dstp, zrows, onesrows)
    d0 = degp[0, :N, 0:1]
    d1 = degp[1, :N, 0:1]

    y1, dinv = _tc_layer1(x, W1, d0, d1)

    acc1 = _sc_scatter(y1, srcp, dstp, zrows)
    y2 = _tc_layer2(acc1[0, :N], acc1[1, :N], y1, dinv,
                    b1.reshape(1, D), W2)

    acc2 = _sc_scatter(y2, srcp, dstp, zrows)
    return _tc_final(acc2[0, :N], acc2[1, :N], y2, dinv, b2.reshape(1, D))


# speedup vs baseline: 23.8991x; 1.2018x over previous
import functools

import jax
import jax.numpy as jnp
from jax import lax
from jax.experimental import pallas as pl
from jax.experimental.pallas import tpu as pltpu
from jax.experimental.pallas import tpu_sc as plsc

N = 10000
D = 128
E = 320000
NC = 2
NS = 16
NW = NC * NS
B = 128
CH = 80
EPW = CH * B
EP = NW * EPW
PADE = EP - E
NR = 10240
STRIPE = NR // NS

_mesh = plsc.VectorSubcoreMesh(core_axis_name="c", subcore_axis_name="s")


def _sc_degree(dstp, zrows, onesrows):
    @functools.partial(
        pl.kernel,
        out_type=jax.ShapeDtypeStruct((NC, NR, D), jnp.float32),
        mesh=_mesh,
        scratch_types=[
            pltpu.VMEM((CH, B), jnp.int32),
            pltpu.VMEM((B, D), jnp.float32),
            pltpu.VMEM_SHARED((NR, D), jnp.float32),
        ],
    )
    def k(dstp_hbm, zeros_hbm, ones_hbm, out_hbm, idx_v, ones_v, degw):
        cid = lax.axis_index("c")
        sid = lax.axis_index("s")
        wid = cid * NS + sid
        row0 = sid * STRIPE
        pltpu.sync_copy(zeros_hbm.at[pl.ds(row0, STRIPE)],
                        degw.at[pl.ds(row0, STRIPE)])
        pltpu.sync_copy(dstp_hbm.at[wid], idx_v)
        pltpu.sync_copy(ones_hbm, ones_v)
        plsc.subcore_barrier()

        def body(ch, carry):
            pltpu.sync_copy(ones_v, degw.at[idx_v.at[ch]], add=True)
            return carry

        lax.fori_loop(0, CH, body, 0)
        plsc.subcore_barrier()
        pltpu.sync_copy(degw.at[pl.ds(row0, STRIPE)],
                        out_hbm.at[cid, pl.ds(row0, STRIPE)])

    return k(dstp, zrows, onesrows)


def _sc_scatter(y, srcp, dstp, zrows):
    @functools.partial(
        pl.kernel,
        out_type=jax.ShapeDtypeStruct((NC, NR, D), jnp.float32),
        mesh=_mesh,
        scratch_types=[
            pltpu.VMEM((CH // 2, B), jnp.int32),
            pltpu.VMEM((CH // 2, B), jnp.int32),
            pltpu.VMEM((B, D), jnp.float32),
            pltpu.VMEM((B, D), jnp.float32),
            pltpu.VMEM_SHARED((NR, D), jnp.float32),
            pltpu.SemaphoreType.DMA,
            pltpu.SemaphoreType.DMA,
        ],
    )
    def k(y_hbm, srcp_hbm, dstp_hbm, z_hbm, out_hbm,
          sidx, didx, rows_a, rows_b, acc, sem_a, sem_b):
        cid = lax.axis_index("c")
        sid = lax.axis_index("s")
        wid = cid * NS + sid
        row0 = sid * STRIPE
        CH2 = CH // 2
        pltpu.sync_copy(z_hbm.at[pl.ds(row0, STRIPE)],
                        acc.at[pl.ds(row0, STRIPE)])
        plsc.subcore_barrier()

        for h in range(2):
            pltpu.sync_copy(srcp_hbm.at[wid, pl.ds(h * CH2, CH2)], sidx)
            pltpu.sync_copy(dstp_hbm.at[wid, pl.ds(h * CH2, CH2)], didx)
            pltpu.async_copy(y_hbm.at[sidx.at[0]], rows_a, sem_a)

            def body(g, carry):
                ca = 2 * g
                pltpu.make_async_copy(y_hbm.at[sidx.at[ca]], rows_a,
                                      sem_a).wait()
                pltpu.async_copy(y_hbm.at[sidx.at[ca + 1]], rows_b, sem_b)
                pltpu.sync_copy(rows_a, acc.at[didx.at[ca]], add=True)
                pltpu.make_async_copy(y_hbm.at[sidx.at[ca + 1]], rows_b,
                                      sem_b).wait()
                nxt = jnp.minimum(ca + 2, CH2 - 1)
                pltpu.async_copy(y_hbm.at[sidx.at[nxt]], rows_a, sem_a)
                pltpu.sync_copy(rows_b, acc.at[didx.at[ca + 1]], add=True)
                return carry

            lax.fori_loop(0, CH2 // 2, body, 0)
            pltpu.make_async_copy(y_hbm.at[sidx.at[CH2 - 1]], rows_a,
                                  sem_a).wait()
        plsc.subcore_barrier()
        pltpu.sync_copy(acc.at[pl.ds(row0, STRIPE)],
                        out_hbm.at[cid, pl.ds(row0, STRIPE)])

    return k(y, srcp, dstp, zrows)


_R = 2000


def _tc_layer1(x, W1, d0, d1):
    def body(x_ref, w_ref, d0_ref, d1_ref, y_ref, dinv_ref):
        dinv = lax.rsqrt(1.0 + d0_ref[...] + d1_ref[...])
        y_ref[...] = jnp.dot(x_ref[...], w_ref[...],
                             preferred_element_type=jnp.float32) * dinv
        dinv_ref[...] = dinv

    return pl.pallas_call(
        body,
        grid=(N // _R,),
        in_specs=[
            pl.BlockSpec((_R, D), lambda i: (i, 0)),
            pl.BlockSpec((D, D), lambda i: (0, 0)),
            pl.BlockSpec((_R, 1), lambda i: (i, 0)),
            pl.BlockSpec((_R, 1), lambda i: (i, 0)),
        ],
        out_specs=[
            pl.BlockSpec((_R, D), lambda i: (i, 0)),
            pl.BlockSpec((_R, 1), lambda i: (i, 0)),
        ],
        out_shape=[
            jax.ShapeDtypeStruct((N, D), jnp.float32),
            jax.ShapeDtypeStruct((N, 1), jnp.float32),
        ],
    )(x, W1, d0, d1)


def _tc_layer2(a0, a1, y1, dinv, b1, W2):
    def body(a0_ref, a1_ref, y1_ref, dinv_ref, b_ref, w_ref, y2_ref):
        dinv = dinv_ref[...]
        h = jnp.maximum(
            dinv * (a0_ref[...] + a1_ref[...] + y1_ref[...]) + b_ref[...],
            0.0)
        y2_ref[...] = jnp.dot(h, w_ref[...],
                              preferred_element_type=jnp.float32) * dinv

    return pl.pallas_call(
        body,
        grid=(N // _R,),
        in_specs=[
            pl.BlockSpec((_R, D), lambda i: (i, 0)),
            pl.BlockSpec((_R, D), lambda i: (i, 0)),
            pl.BlockSpec((_R, D), lambda i: (i, 0)),
            pl.BlockSpec((_R, 1), lambda i: (i, 0)),
            pl.BlockSpec((1, D), lambda i: (0, 0)),
            pl.BlockSpec((D, D), lambda i: (0, 0)),
        ],
        out_specs=pl.BlockSpec((_R, D), lambda i: (i, 0)),
        out_shape=jax.ShapeDtypeStruct((N, D), jnp.float32),
    )(a0, a1, y1, dinv, b1, W2)


def _tc_final(a0, a1, y2, dinv, b2):
    def body(a0_ref, a1_ref, y2_ref, dinv_ref, b_ref, out_ref):
        out_ref[...] = jnp.maximum(
            dinv_ref[...] * (a0_ref[...] + a1_ref[...] + y2_ref[...])
            + b_ref[...], 0.0)

    return pl.pallas_call(
        body,
        grid=(N // _R,),
        in_specs=[
            pl.BlockSpec((_R, D), lambda i: (i, 0)),
            pl.BlockSpec((_R, D), lambda i: (i, 0)),
            pl.BlockSpec((_R, D), lambda i: (i, 0)),
            pl.BlockSpec((_R, 1), lambda i: (i, 0)),
            pl.BlockSpec((1, D), lambda i: (0, 0)),
        ],
        out_specs=pl.BlockSpec((_R, D), lambda i: (i, 0)),
        out_shape=jax.ShapeDtypeStruct((N, D), jnp.float32),
    )(a0, a1, y2, dinv, b2)


def kernel(x, edge_index, W1, b1, W2, b2):
    src = edge_index[0].astype(jnp.int32)
    dst = edge_index[1].astype(jnp.int32)

    pad_src = (jnp.arange(PADE, dtype=jnp.int32) * 13) % N
    pad_dst = N + (jnp.arange(PADE, dtype=jnp.int32) % (NR - N))
    srcp = jnp.concatenate([src, pad_src]).reshape(NW, CH, B)
    dstp = jnp.concatenate([dst, pad_dst]).reshape(NW, CH, B)

    onesrows = jnp.ones((B, D), jnp.float32)
    zrows = jnp.zeros((NR, D), jnp.float32)

    degp = _sc_degree(dstp, zrows, onesrows)
    d0 = degp[0, :N, 0:1]
    d1 = degp[1, :N, 0:1]

    y1, dinv = _tc_layer1(x, W1, d0, d1)

    acc1 = _sc_scatter(y1, srcp, dstp, zrows)
    y2 = _tc_layer2(acc1[0, :N], acc1[1, :N], y1, dinv,
                    b1.reshape(1, D), W2)

    acc2 = _sc_scatter(y2, srcp, dstp, zrows)
    return _tc_final(acc2[0, :N], acc2[1, :N], y2, dinv, b2.reshape(1, D))


# speedup vs baseline: 23.9638x; 1.0027x over previous
import functools

import jax
import jax.numpy as jnp
import numpy as np
from jax import lax
from jax.experimental import pallas as pl
from jax.experimental.pallas import tpu as pltpu
from jax.experimental.pallas import tpu_sc as plsc

N = 10000
D = 128
E = 320000
NC = 2
NS = 16
NW = NC * NS
B = 128
CH = 80
EPW = CH * B
EP = NW * EPW
PADE = EP - E
NR = 10240
STRIPE = NR // NS

_mesh = plsc.VectorSubcoreMesh(core_axis_name="c", subcore_axis_name="s")


def _sc_degree(dstp, zrows, onesrows):
    @functools.partial(
        pl.kernel,
        out_type=jax.ShapeDtypeStruct((NC, NR, D), jnp.float32),
        mesh=_mesh,
        scratch_types=[
            pltpu.VMEM((CH, B), jnp.int32),
            pltpu.VMEM((B, D), jnp.float32),
            pltpu.VMEM_SHARED((NR, D), jnp.float32),
        ],
    )
    def k(dstp_hbm, zeros_hbm, ones_hbm, out_hbm, idx_v, ones_v, degw):
        cid = lax.axis_index("c")
        sid = lax.axis_index("s")
        wid = cid * NS + sid
        row0 = sid * STRIPE
        pltpu.sync_copy(zeros_hbm.at[pl.ds(row0, STRIPE)],
                        degw.at[pl.ds(row0, STRIPE)])
        pltpu.sync_copy(dstp_hbm.at[wid], idx_v)
        pltpu.sync_copy(ones_hbm, ones_v)
        plsc.subcore_barrier()

        def body(ch, carry):
            pltpu.sync_copy(ones_v, degw.at[idx_v.at[ch]], add=True)
            return carry

        lax.fori_loop(0, CH, body, 0)
        plsc.subcore_barrier()
        pltpu.sync_copy(degw.at[pl.ds(row0, STRIPE)],
                        out_hbm.at[cid, pl.ds(row0, STRIPE)])

    return k(dstp, zrows, onesrows)


def _sc_scatter(y, srcp, dstp, zrows):
    @functools.partial(
        pl.kernel,
        out_type=jax.ShapeDtypeStruct((NC, NR, D), jnp.float32),
        mesh=_mesh,
        scratch_types=[
            pltpu.VMEM((CH // 2, B), jnp.int32),
            pltpu.VMEM((CH // 2, B), jnp.int32),
            pltpu.VMEM((B, D), jnp.float32),
            pltpu.VMEM((B, D), jnp.float32),
            pltpu.VMEM_SHARED((NR, D), jnp.float32),
            pltpu.SemaphoreType.DMA,
            pltpu.SemaphoreType.DMA,
        ],
    )
    def k(y_hbm, srcp_hbm, dstp_hbm, z_hbm, out_hbm,
          sidx, didx, rows_a, rows_b, acc, sem_a, sem_b):
        cid = lax.axis_index("c")
        sid = lax.axis_index("s")
        wid = cid * NS + sid
        row0 = sid * STRIPE
        CH2 = CH // 2
        pltpu.sync_copy(z_hbm.at[pl.ds(row0, STRIPE)],
                        acc.at[pl.ds(row0, STRIPE)])
        plsc.subcore_barrier()

        for h in range(2):
            pltpu.sync_copy(srcp_hbm.at[wid, pl.ds(h * CH2, CH2)], sidx)
            pltpu.sync_copy(dstp_hbm.at[wid, pl.ds(h * CH2, CH2)], didx)
            pltpu.async_copy(y_hbm.at[sidx.at[0]], rows_a, sem_a)

            def body(g, carry):
                ca = 2 * g
                pltpu.make_async_copy(y_hbm.at[sidx.at[ca]], rows_a,
                                      sem_a).wait()
                pltpu.async_copy(y_hbm.at[sidx.at[ca + 1]], rows_b, sem_b)
                pltpu.sync_copy(rows_a, acc.at[didx.at[ca]], add=True)
                pltpu.make_async_copy(y_hbm.at[sidx.at[ca + 1]], rows_b,
                                      sem_b).wait()
                nxt = jnp.minimum(ca + 2, CH2 - 1)
                pltpu.async_copy(y_hbm.at[sidx.at[nxt]], rows_a, sem_a)
                pltpu.sync_copy(rows_b, acc.at[didx.at[ca + 1]], add=True)
                return carry

            lax.fori_loop(0, CH2 // 2, body, 0)
            pltpu.make_async_copy(y_hbm.at[sidx.at[CH2 - 1]], rows_a,
                                  sem_a).wait()
        plsc.subcore_barrier()
        pltpu.sync_copy(acc.at[pl.ds(row0, STRIPE)],
                        out_hbm.at[cid, pl.ds(row0, STRIPE)])

    return k(y, srcp, dstp, zrows)


_R = 2000


def _tc_layer1(x, W1, d0, d1):
    def body(x_ref, w_ref, d0_ref, d1_ref, y_ref, dinv_ref):
        dinv = lax.rsqrt(1.0 + d0_ref[...] + d1_ref[...])
        y_ref[...] = jnp.dot(x_ref[...], w_ref[...],
                             preferred_element_type=jnp.float32) * dinv
        dinv_ref[...] = dinv

    return pl.pallas_call(
        body,
        grid=(N // _R,),
        in_specs=[
            pl.BlockSpec((_R, D), lambda i: (i, 0)),
            pl.BlockSpec((D, D), lambda i: (0, 0)),
            pl.BlockSpec((_R, 1), lambda i: (i, 0)),
            pl.BlockSpec((_R, 1), lambda i: (i, 0)),
        ],
        out_specs=[
            pl.BlockSpec((_R, D), lambda i: (i, 0)),
            pl.BlockSpec((_R, 1), lambda i: (i, 0)),
        ],
        out_shape=[
            jax.ShapeDtypeStruct((N, D), jnp.float32),
            jax.ShapeDtypeStruct((N, 1), jnp.float32),
        ],
    )(x, W1, d0, d1)


def _tc_layer2(a0, a1, y1, dinv, b1, W2):
    def body(a0_ref, a1_ref, y1_ref, dinv_ref, b_ref, w_ref, y2_ref):
        dinv = dinv_ref[...]
        h = jnp.maximum(
            dinv * (a0_ref[...] + a1_ref[...] + y1_ref[...]) + b_ref[...],
            0.0)
        y2_ref[...] = jnp.dot(h, w_ref[...],
                              preferred_element_type=jnp.float32) * dinv

    return pl.pallas_call(
        body,
        grid=(N // _R,),
        in_specs=[
            pl.BlockSpec((_R, D), lambda i: (i, 0)),
            pl.BlockSpec((_R, D), lambda i: (i, 0)),
            pl.BlockSpec((_R, D), lambda i: (i, 0)),
            pl.BlockSpec((_R, 1), lambda i: (i, 0)),
            pl.BlockSpec((1, D), lambda i: (0, 0)),
            pl.BlockSpec((D, D), lambda i: (0, 0)),
        ],
        out_specs=pl.BlockSpec((_R, D), lambda i: (i, 0)),
        out_shape=jax.ShapeDtypeStruct((N, D), jnp.float32),
    )(a0, a1, y1, dinv, b1, W2)


def _tc_final(a0, a1, y2, dinv, b2):
    def body(a0_ref, a1_ref, y2_ref, dinv_ref, b_ref, out_ref):
        out_ref[...] = jnp.maximum(
            dinv_ref[...] * (a0_ref[...] + a1_ref[...] + y2_ref[...])
            + b_ref[...], 0.0)

    return pl.pallas_call(
        body,
        grid=(N // _R,),
        in_specs=[
            pl.BlockSpec((_R, D), lambda i: (i, 0)),
            pl.BlockSpec((_R, D), lambda i: (i, 0)),
            pl.BlockSpec((_R, D), lambda i: (i, 0)),
            pl.BlockSpec((_R, 1), lambda i: (i, 0)),
            pl.BlockSpec((1, D), lambda i: (0, 0)),
        ],
        out_specs=pl.BlockSpec((_R, D), lambda i: (i, 0)),
        out_shape=jax.ShapeDtypeStruct((N, D), jnp.float32),
    )(a0, a1, y2, dinv, b2)


_PAD_SRC = np.asarray((np.arange(PADE) * 13) % N, np.int32)
_PAD_DST = np.asarray(N + np.arange(PADE) % (NR - N), np.int32)


def kernel(x, edge_index, W1, b1, W2, b2):
    src = edge_index[0].astype(jnp.int32)
    dst = edge_index[1].astype(jnp.int32)

    srcp = jnp.concatenate([src, jnp.asarray(_PAD_SRC)]).reshape(NW, CH, B)
    dstp = jnp.concatenate([dst, jnp.asarray(_PAD_DST)]).reshape(NW, CH, B)

    onesrows = jnp.ones((B, D), jnp.float32)
    zrows = jnp.zeros((NR, D), jnp.float32)

    degp = _sc_degree(dstp, zrows, onesrows)
    d0 = degp[0, :N, 0:1]
    d1 = degp[1, :N, 0:1]

    y1, dinv = _tc_layer1(x, W1, d0, d1)

    acc1 = _sc_scatter(y1, srcp, dstp, zrows)
    y2 = _tc_layer2(acc1[0, :N], acc1[1, :N], y1, dinv,
                    b1.reshape(1, D), W2)

    acc2 = _sc_scatter(y2, srcp, dstp, zrows)
    return _tc_final(acc2[0, :N], acc2[1, :N], y2, dinv, b2.reshape(1, D))
